# baseline (speedup 1.0000x reference)
import jax
import jax.numpy as jnp
from jax import lax
from jax.experimental import pallas as pl
from jax.experimental.pallas import tpu as pltpu
from jax.experimental.pallas import tpu_sc as plsc

B = 512
G = 978
N = B * G
E = N * 16
N_TILES = 16
E_PER_TILE = E // N_TILES
CHUNK = 10432
N_CHUNKS = E_PER_TILE // CHUNK
N_PER_TILE = N // N_TILES
STAGE = N_PER_TILE // 4


def _sc_segment_sum(x1, x2, edges_r, zeros_n):
    mesh = plsc.VectorSubcoreMesh(core_axis_name="c", subcore_axis_name="s")

    def body(x1_hbm, x2_hbm, edges_hbm, zeros_hbm, out_hbm,
             xsh, aggsh, s0, s1, d0, d1, v0, v1,
             sem_s0, sem_s1, sem_d0, sem_d1, sem_g0, sem_g1,
             sem_c0, sem_c1):
        c = lax.axis_index("c")
        s = lax.axis_index("s")
        n0 = s * N_PER_TILE

        stage = v0.at[pl.ds(0, STAGE)]
        for k in range(N_PER_TILE // STAGE):
            p0 = n0 + k * STAGE

            @pl.when(c == 0)
            def _():
                pltpu.sync_copy(x1_hbm.at[pl.ds(p0, STAGE)], stage)

            @pl.when(c != 0)
            def _():
                pltpu.sync_copy(x2_hbm.at[pl.ds(p0, STAGE)], stage)

            pltpu.sync_copy(stage, xsh.at[pl.ds(p0, STAGE)])
            pltpu.sync_copy(zeros_hbm.at[pl.ds(p0, STAGE)], stage)
            pltpu.sync_copy(stage, aggsh.at[pl.ds(p0, STAGE)])
        plsc.subcore_barrier()

        tile_e0 = s * E_PER_TILE

        def edge_dma(i, sb, db, sem_sb, sem_db):
            e0 = tile_e0 + i * CHUNK
            pltpu.async_copy(edges_hbm.at[pl.ds(e0, CHUNK)], sb, sem_sb)
            pltpu.async_copy(edges_hbm.at[pl.ds(E + e0, CHUNK)], db, sem_db)

        def edge_wait(i, sb, db, sem_sb, sem_db):
            e0 = tile_e0 + i * CHUNK
            pltpu.make_async_copy(edges_hbm.at[pl.ds(e0, CHUNK)], sb,
                                  sem_sb).wait()
            pltpu.make_async_copy(edges_hbm.at[pl.ds(E + e0, CHUNK)], db,
                                  sem_db).wait()

        edge_dma(0, s0, d0, sem_s0, sem_d0)
        edge_dma(1, s1, d1, sem_s1, sem_d1)

        def step(k, carry):
            a = 2 * k
            edge_wait(a, s0, d0, sem_s0, sem_d0)
            g_a = pltpu.async_copy(xsh.at[s0], v0, sem_g0)
            edge_wait(a + 1, s1, d1, sem_s1, sem_d1)
            g_b = pltpu.async_copy(xsh.at[s1], v1, sem_g1)
            g_a.wait()
            c_a = pltpu.async_copy(v0, aggsh.at[d0], sem_c0, add=True)

            @pl.when(k < (N_CHUNKS // 2) - 1)
            def _():
                e0 = tile_e0 + (a + 2) * CHUNK
                pltpu.async_copy(edges_hbm.at[pl.ds(e0, CHUNK)], s0, sem_s0)

            g_b.wait()
            c_b = pltpu.async_copy(v1, aggsh.at[d1], sem_c1, add=True)

            @pl.when(k < (N_CHUNKS // 2) - 1)
            def _():
                e0 = tile_e0 + (a + 3) * CHUNK
                pltpu.async_copy(edges_hbm.at[pl.ds(e0, CHUNK)], s1, sem_s1)

            c_a.wait()

            @pl.when(k < (N_CHUNKS // 2) - 1)
            def _():
                e0 = tile_e0 + (a + 2) * CHUNK
                pltpu.async_copy(edges_hbm.at[pl.ds(E + e0, CHUNK)], d0,
                                 sem_d0)

            c_b.wait()

            @pl.when(k < (N_CHUNKS // 2) - 1)
            def _():
                e0 = tile_e0 + (a + 3) * CHUNK
                pltpu.async_copy(edges_hbm.at[pl.ds(E + e0, CHUNK)], d1,
                                 sem_d1)

            return carry

        lax.fori_loop(0, N_CHUNKS // 2, step, 0)
        plsc.subcore_barrier()
        for k in range(N_PER_TILE // STAGE):
            p0 = n0 + k * STAGE
            pltpu.sync_copy(aggsh.at[pl.ds(p0, STAGE)], stage)
            pltpu.sync_copy(stage, out_hbm.at[pl.ds(c * N + p0, STAGE)])

    f = pl.kernel(
        body,
        out_type=jax.ShapeDtypeStruct((2 * N,), jnp.float32),
        mesh=mesh,
        scratch_types=[
            pltpu.VMEM_SHARED((N,), jnp.float32),
            pltpu.VMEM_SHARED((N,), jnp.float32),
            pltpu.VMEM((CHUNK,), jnp.int32),
            pltpu.VMEM((CHUNK,), jnp.int32),
            pltpu.VMEM((CHUNK,), jnp.int32),
            pltpu.VMEM((CHUNK,), jnp.int32),
            pltpu.VMEM((CHUNK,), jnp.float32),
            pltpu.VMEM((CHUNK,), jnp.float32),
        ] + [pltpu.SemaphoreType.DMA] * 8,
    )
    return f(x1, x2, edges_r, zeros_n)


def _tc_head(agg, others, gcn_w, gcn_b, fc1_w, fc1_b, fc2_w, fc2_b,
             mlp1_w, mlp1_b, mlp2_w, mlp2_b, interpret=False):
    def body(agg_ref, oth_ref, gw_ref, gb_ref, w1_ref, b1_ref, w2_ref, b2_ref,
             m1w_ref, m1b_ref, m2w_ref, m2b_ref, out_ref):
        gw = gw_ref[...]
        gb = gb_ref[...]

        def enc(a):
            h = jnp.maximum(a * gw + gb, 0.0)
            h = lax.dot_general(h, w1_ref[...], (((1,), (1,)), ((), ())),
                                preferred_element_type=jnp.float32)
            h = jnp.maximum(h + b1_ref[...], 0.0)
            o = lax.dot_general(h, w2_ref[...], (((1,), (1,)), ((), ())),
                                preferred_element_type=jnp.float32)
            return o + b2_ref[...]

        o1 = enc(agg_ref[0])
        o2 = enc(agg_ref[1])
        p1 = o1 - jnp.mean(o1, axis=1, keepdims=True)
        p2 = o2 - jnp.mean(o2, axis=1, keepdims=True)
        n1 = jnp.sum(p1 * p1, axis=1, keepdims=True)
        n2 = jnp.sum(p2 * p2, axis=1, keepdims=True)
        p12 = jnp.sum(p1 * p2, axis=1, keepdims=True)
        r = p12 / jnp.sqrt(n1 * n2)
        r2 = r * r
        cat = jnp.concatenate([r2, oth_ref[...]], axis=1)
        z = lax.dot_general(cat, m1w_ref[...], (((1,), (1,)), ((), ())),
                            preferred_element_type=jnp.float32)
        z = jnp.maximum(z + m1b_ref[...], 0.0)
        out = lax.dot_general(z, m2w_ref[...], (((1,), (1,)), ((), ())),
                              preferred_element_type=jnp.float32)
        out_ref[...] = out + m2b_ref[...]

    return pl.pallas_call(
        body,
        out_shape=jax.ShapeDtypeStruct((B, 2), jnp.float32),
        interpret=interpret,
    )(agg, others, gcn_w, gcn_b, fc1_w, fc1_b, fc2_w, fc2_b,
      mlp1_w, mlp1_b, mlp2_w, mlp2_b)


def kernel(input1, input2, edges, input_others, gcn_w, gcn_b,
           fc1_w, fc1_b, fc2_w, fc2_b, mlp1_w, mlp1_b, mlp2_w, mlp2_b):
    x1 = input1.reshape(-1)
    x2 = input2.reshape(-1)
    zeros_n = jnp.zeros((N,), jnp.float32)
    agg = _sc_segment_sum(x1, x2, edges.reshape(2 * E), zeros_n)
    return _tc_head(agg.reshape(2, B, G), input_others,
                    gcn_w, gcn_b.reshape(1, 1),
                    fc1_w, fc1_b.reshape(1, -1),
                    fc2_w, fc2_b.reshape(1, -1),
                    mlp1_w, mlp1_b.reshape(1, -1),
                    mlp2_w, mlp2_b.reshape(1, -1))
